# baseline (speedup 1.0000x reference)
import functools

import jax
import jax.numpy as jnp
from jax import lax
from jax.experimental import pallas as pl
from jax.experimental.pallas import tpu as pltpu
from jax.experimental.pallas import tpu_sc as plsc

N = 10000
NPAD = 10240
D = 128
HID = 256
NEDGE = 320000
NC = 2
NS = 16
EPT = NEDGE // (NC * NS)
K = 128
NFULL = EPT // K
REM = EPT - NFULL * K
EPT2 = NEDGE // NS
NFULL2 = EPT2 // K
REM2 = EPT2 - NFULL2 * K
ROWS_PER_TILE = NPAD // NS

_mesh = plsc.VectorSubcoreMesh(core_axis_name="c", subcore_axis_name="s")


@functools.partial(
    pl.kernel,
    out_type=jax.ShapeDtypeStruct((NC, 2, NPAD), jnp.float32),
    mesh=_mesh,
    scratch_types=[
        pltpu.VMEM((K,), jnp.int32),
        pltpu.VMEM((K,), jnp.int32),
        pltpu.VMEM((REM,), jnp.int32),
        pltpu.VMEM((K,), jnp.float32),
        pltpu.VMEM((REM,), jnp.float32),
        pltpu.VMEM((ROWS_PER_TILE,), jnp.float32),
        pltpu.VMEM_SHARED((NPAD,), jnp.float32),
        pltpu.VMEM_SHARED((NPAD,), jnp.float32),
        pltpu.SemaphoreType.DMA,
        pltpu.SemaphoreType.DMA,
    ],
)
def _deg_fn(i0, i1, out, idxa, idxb, idxr, onev, oner, zv, h0, h1, sa, sb):
    c = lax.axis_index("c")
    s = lax.axis_index("s")
    for i in range(K // 16):
        onev[pl.ds(i * 16, 16)] = jnp.ones((16,), jnp.float32)
    oner[...] = jnp.ones((REM,), jnp.float32)
    for i in range(ROWS_PER_TILE // 16):
        zv[pl.ds(i * 16, 16)] = jnp.zeros((16,), jnp.float32)
    for h in (h0, h1):
        pltpu.sync_copy(zv, h.at[pl.ds(s * ROWS_PER_TILE, ROWS_PER_TILE)])
    plsc.subcore_barrier()
    base0 = (c * NS + s) * EPT
    for src, h in ((i0, h0), (i1, h1)):
        def body(i, _, src=src, h=h):
            b = base0 + 2 * i * K
            cpa = pltpu.async_copy(src.at[pl.ds(b, K)], idxa, sa)
            cpb = pltpu.async_copy(src.at[pl.ds(b + K, K)], idxb, sb)
            cpa.wait()
            pltpu.sync_copy(onev, h.at[idxa], add=True)
            cpb.wait()
            pltpu.sync_copy(onev, h.at[idxb], add=True)
            return 0
        lax.fori_loop(0, NFULL // 2, body, 0)
        b = base0 + NFULL * K
        pltpu.sync_copy(src.at[pl.ds(b, REM)], idxr)
        pltpu.sync_copy(oner, h.at[idxr], add=True)
    plsc.subcore_barrier()
    sl = pl.ds(s * ROWS_PER_TILE, ROWS_PER_TILE)
    for r, h in enumerate((h0, h1)):
        pltpu.sync_copy(h.at[sl], out.at[c, r, sl])


@functools.partial(
    pl.kernel,
    out_type=[
        jax.ShapeDtypeStruct((NC, NPAD, D), jnp.float32),
        jax.ShapeDtypeStruct((NC, NPAD), jnp.float32),
    ],
    mesh=_mesh,
    scratch_types=[
        pltpu.VMEM((K,), jnp.int32),
        pltpu.VMEM((K,), jnp.int32),
        pltpu.VMEM((K,), jnp.int32),
        pltpu.VMEM((K,), jnp.int32),
        pltpu.VMEM((REM2,), jnp.int32),
        pltpu.VMEM((REM2,), jnp.int32),
        pltpu.VMEM((K, D), jnp.float32),
        pltpu.VMEM((K, D), jnp.float32),
        pltpu.VMEM((REM2, D), jnp.float32),
        pltpu.VMEM((K,), jnp.float32),
        pltpu.VMEM((REM2,), jnp.float32),
        pltpu.VMEM((16, D), jnp.float32),
        pltpu.VMEM((ROWS_PER_TILE,), jnp.float32),
        pltpu.VMEM_SHARED((NPAD, D), jnp.float32),
        pltpu.VMEM_SHARED((NPAD,), jnp.float32),
        pltpu.SemaphoreType.DMA,
        pltpu.SemaphoreType.DMA,
    ],
)
def _spmm_fn(xn0, src0, dst0, xn1, src1, dst1, out, dout,
             sia, dia, sib, dib, sidr, didr,
             rowsa, rowsb, rowr, onev, oner, zb, zv, acc, hdeg, sema, semb):
    c = lax.axis_index("c")
    s = lax.axis_index("s")
    for i in range(16):
        for j in range(D // 16):
            zb[i, pl.ds(j * 16, 16)] = jnp.zeros((16,), jnp.float32)
    for i in range(K // 16):
        onev[pl.ds(i * 16, 16)] = jnp.ones((16,), jnp.float32)
    oner[...] = jnp.ones((REM2,), jnp.float32)
    for i in range(ROWS_PER_TILE // 16):
        zv[pl.ds(i * 16, 16)] = jnp.zeros((16,), jnp.float32)
    for k in range(ROWS_PER_TILE // 16):
        pltpu.sync_copy(zb, acc.at[pl.ds(s * ROWS_PER_TILE + k * 16, 16)])
    pltpu.sync_copy(zv, hdeg.at[pl.ds(s * ROWS_PER_TILE, ROWS_PER_TILE)])
    plsc.subcore_barrier()
    base0 = s * EPT2

    def run(xn, src, dst):
        def body(i, _):
            ba = base0 + 2 * i * K
            bb = ba + K
            pltpu.sync_copy(src.at[pl.ds(ba, K)], sia)
            pltpu.sync_copy(dst.at[pl.ds(ba, K)], dia)
            cpa = pltpu.async_copy(xn.at[sia], rowsa, sema)
            pltpu.sync_copy(src.at[pl.ds(bb, K)], sib)
            pltpu.sync_copy(dst.at[pl.ds(bb, K)], dib)
            cpb = pltpu.async_copy(xn.at[sib], rowsb, semb)
            pltpu.sync_copy(onev, hdeg.at[dia], add=True)
            cpa.wait()
            pltpu.sync_copy(rowsa, acc.at[dia], add=True)
            pltpu.sync_copy(onev, hdeg.at[dib], add=True)
            cpb.wait()
            pltpu.sync_copy(rowsb, acc.at[dib], add=True)
            return 0

        lax.fori_loop(0, NFULL2 // 2, body, 0)
        b = base0 + NFULL2 * K
        pltpu.sync_copy(src.at[pl.ds(b, REM2)], sidr)
        pltpu.sync_copy(dst.at[pl.ds(b, REM2)], didr)
        cpr = pltpu.async_copy(xn.at[sidr], rowr, sema)
        pltpu.sync_copy(oner, hdeg.at[didr], add=True)
        cpr.wait()
        pltpu.sync_copy(rowr, acc.at[didr], add=True)

    @pl.when(c == 0)
    def _():
        run(xn0, src0, dst0)

    @pl.when(c == 1)
    def _():
        run(xn1, src1, dst1)

    plsc.subcore_barrier()
    sl = pl.ds(s * ROWS_PER_TILE, ROWS_PER_TILE)
    pltpu.sync_copy(acc.at[sl], out.at[c, sl])
    pltpu.sync_copy(hdeg.at[sl], dout.at[c, sl])


_R = 1024
_GRID = NPAD // _R


def _pre_body(g_ref, b_ref, wres_ref, wrel_ref, gate_ref, h_ref, deg_ref,
              proj_ref, xn_ref):
    x = h_ref[...]
    m = jnp.mean(x, axis=-1, keepdims=True)
    v = jnp.mean((x - m) * (x - m), axis=-1, keepdims=True)
    ln = (x - m) * lax.rsqrt(v + 1e-5) * g_ref[0, :] + b_ref[0, :]
    proj_ref[...] = jnp.dot(ln, wres_ref[...],
                            preferred_element_type=jnp.float32)
    deg = deg_ref[0, :] + deg_ref[1, :]
    sc = jnp.where(deg > 0.0, lax.rsqrt(jnp.maximum(deg, 1.0)), 0.0)
    sc = sc * gate_ref[0, 0]
    xw = jnp.dot(x, wrel_ref[...], preferred_element_type=jnp.float32)
    xn_ref[...] = xw * sc[:, None]


_pre_call = pl.pallas_call(
    _pre_body,
    grid=(_GRID,),
    in_specs=[
        pl.BlockSpec((1, D), lambda i: (0, 0)),
        pl.BlockSpec((1, D), lambda i: (0, 0)),
        pl.BlockSpec((D, D), lambda i: (0, 0)),
        pl.BlockSpec((D, D), lambda i: (0, 0)),
        pl.BlockSpec(memory_space=pltpu.SMEM),
        pl.BlockSpec((_R, D), lambda i: (i, 0)),
        pl.BlockSpec((2, _R), lambda i: (0, i)),
    ],
    out_specs=[
        pl.BlockSpec((_R, D), lambda i: (i, 0)),
        pl.BlockSpec((_R, D), lambda i: (i, 0)),
    ],
    out_shape=[
        jax.ShapeDtypeStruct((NPAD, D), jnp.float32),
        jax.ShapeDtypeStruct((NPAD, D), jnp.float32),
    ],
)


def _upd_body(w1_ref, b1_ref, w2_ref, b2_ref, proj_ref, part_ref, deg_ref,
              out_ref):
    proj = proj_ref[...]
    deg = deg_ref[0, :]
    sc = jnp.where(deg > 0.0, lax.rsqrt(jnp.maximum(deg, 1.0)), 0.0)
    agg = part_ref[...] * sc[:, None]
    u = jnp.concatenate([proj, agg], axis=-1)
    hpre = jnp.dot(u, w1_ref[...], preferred_element_type=jnp.float32)
    hpre = hpre + b1_ref[0, :]
    h = hpre * 0.5 * (1.0 + lax.erf(hpre * 0.7071067811865476))
    out = jnp.dot(h, w2_ref[...], preferred_element_type=jnp.float32)
    out_ref[...] = proj + out + b2_ref[0, :]


_upd_call = pl.pallas_call(
    _upd_body,
    grid=(_GRID,),
    in_specs=[
        pl.BlockSpec((2 * D, HID), lambda i: (0, 0)),
        pl.BlockSpec((1, HID), lambda i: (0, 0)),
        pl.BlockSpec((HID, D), lambda i: (0, 0)),
        pl.BlockSpec((1, D), lambda i: (0, 0)),
        pl.BlockSpec((_R, D), lambda i: (i, 0)),
        pl.BlockSpec((_R, D), lambda i: (i, 0)),
        pl.BlockSpec((1, _R), lambda i: (0, i)),
    ],
    out_specs=pl.BlockSpec((_R, D), lambda i: (i, 0)),
    out_shape=jax.ShapeDtypeStruct((NPAD, D), jnp.float32),
)


def kernel(h_V, h_E, edge_index_V_E, edge_index_E_V,
           ln_g_V, ln_b_V, W_res_V, W1_V, b1_V, W2_V, b2_V,
           ln_g_E, ln_b_E, W_res_E, W1_E, b1_E, W2_E, b2_E,
           W_rel_VE, W_rel_EV, gate_VE, gate_EV):
    src_VE = edge_index_V_E[0].astype(jnp.int32)
    dst_VE = edge_index_V_E[1].astype(jnp.int32)
    src_EV = edge_index_E_V[0].astype(jnp.int32)
    dst_EV = edge_index_E_V[1].astype(jnp.int32)

    degp = _deg_fn(src_VE, src_EV)

    pad = ((0, NPAD - N), (0, 0))
    hV = jnp.pad(h_V, pad)
    hE = jnp.pad(h_E, pad)
    g2 = lambda a: a.reshape(1, -1)
    gateVE = jnp.reshape(gate_VE, (1, 1))
    gateEV = jnp.reshape(gate_EV, (1, 1))

    proj_V, xn_VE = _pre_call(g2(ln_g_V), g2(ln_b_V), W_res_V, W_rel_VE,
                              gateVE, hV, degp[:, 0, :])
    proj_E, xn_EV = _pre_call(g2(ln_g_E), g2(ln_b_E), W_res_E, W_rel_EV,
                              gateEV, hE, degp[:, 1, :])

    part, degd = _spmm_fn(xn_VE, src_VE, dst_VE, xn_EV, src_EV, dst_EV)

    out_V = _upd_call(W1_V, g2(b1_V), W2_V, g2(b2_V), proj_V, part[1],
                      degd[1:2])
    out_E = _upd_call(W1_E, g2(b1_E), W2_E, g2(b2_E), proj_E, part[0],
                      degd[0:1])
    return (out_V[:N], out_E[:N])


# speedup vs baseline: 1.0278x; 1.0278x over previous
import functools

import jax
import jax.numpy as jnp
from jax import lax
from jax.experimental import pallas as pl
from jax.experimental.pallas import tpu as pltpu
from jax.experimental.pallas import tpu_sc as plsc

N = 10000
NPAD = 10240
D = 128
HID = 256
NEDGE = 320000
NC = 2
NS = 16
EPT = NEDGE // (NC * NS)
K = 128
NFULL = EPT // K
REM = EPT - NFULL * K
ROWS_PER_TILE = NPAD // NS

_mesh = plsc.VectorSubcoreMesh(core_axis_name="c", subcore_axis_name="s")


@functools.partial(
    pl.kernel,
    out_type=jax.ShapeDtypeStruct((NC, 2, NPAD), jnp.float32),
    mesh=_mesh,
    scratch_types=[
        pltpu.VMEM((K,), jnp.int32),
        pltpu.VMEM((K,), jnp.int32),
        pltpu.VMEM((REM,), jnp.int32),
        pltpu.VMEM((K,), jnp.float32),
        pltpu.VMEM((REM,), jnp.float32),
        pltpu.VMEM((ROWS_PER_TILE,), jnp.float32),
        pltpu.VMEM_SHARED((NPAD,), jnp.float32),
        pltpu.VMEM_SHARED((NPAD,), jnp.float32),
        pltpu.SemaphoreType.DMA,
        pltpu.SemaphoreType.DMA,
    ],
)
def _deg_fn(i0, i1, out, idxa, idxb, idxr, onev, oner, zv, h0, h1, sa, sb):
    c = lax.axis_index("c")
    s = lax.axis_index("s")
    for i in range(K // 16):
        onev[pl.ds(i * 16, 16)] = jnp.ones((16,), jnp.float32)
    oner[...] = jnp.ones((REM,), jnp.float32)
    for i in range(ROWS_PER_TILE // 16):
        zv[pl.ds(i * 16, 16)] = jnp.zeros((16,), jnp.float32)
    for h in (h0, h1):
        pltpu.sync_copy(zv, h.at[pl.ds(s * ROWS_PER_TILE, ROWS_PER_TILE)])
    plsc.subcore_barrier()
    base0 = (c * NS + s) * EPT
    for src, h in ((i0, h0), (i1, h1)):
        def body(i, _, src=src, h=h):
            b = base0 + 2 * i * K
            cpa = pltpu.async_copy(src.at[pl.ds(b, K)], idxa, sa)
            cpb = pltpu.async_copy(src.at[pl.ds(b + K, K)], idxb, sb)
            cpa.wait()
            pltpu.sync_copy(onev, h.at[idxa], add=True)
            cpb.wait()
            pltpu.sync_copy(onev, h.at[idxb], add=True)
            return 0
        lax.fori_loop(0, NFULL // 2, body, 0)
        b = base0 + NFULL * K
        pltpu.sync_copy(src.at[pl.ds(b, REM)], idxr)
        pltpu.sync_copy(oner, h.at[idxr], add=True)
    plsc.subcore_barrier()
    sl = pl.ds(s * ROWS_PER_TILE, ROWS_PER_TILE)
    for r, h in enumerate((h0, h1)):
        pltpu.sync_copy(h.at[sl], out.at[c, r, sl])


@functools.partial(
    pl.kernel,
    out_type=[
        jax.ShapeDtypeStruct((NC, NPAD, D), jnp.float32),
        jax.ShapeDtypeStruct((NC, NPAD), jnp.float32),
    ],
    mesh=_mesh,
    scratch_types=[
        pltpu.VMEM((K,), jnp.int32),
        pltpu.VMEM((K,), jnp.int32),
        pltpu.VMEM((K,), jnp.int32),
        pltpu.VMEM((K,), jnp.int32),
        pltpu.VMEM((REM,), jnp.int32),
        pltpu.VMEM((REM,), jnp.int32),
        pltpu.VMEM((K, D), jnp.float32),
        pltpu.VMEM((K, D), jnp.float32),
        pltpu.VMEM((REM, D), jnp.float32),
        pltpu.VMEM((K,), jnp.float32),
        pltpu.VMEM((REM,), jnp.float32),
        pltpu.VMEM((16, D), jnp.float32),
        pltpu.VMEM((ROWS_PER_TILE,), jnp.float32),
        pltpu.VMEM_SHARED((NPAD, D), jnp.float32),
        pltpu.VMEM_SHARED((NPAD,), jnp.float32),
        pltpu.SemaphoreType.DMA,
        pltpu.SemaphoreType.DMA,
    ],
)
def _spmm_fn(xn, src, dst, out, dout,
             sia, dia, sib, dib, sidr, didr,
             rowsa, rowsb, rowr, onev, oner, zb, zv, acc, hdeg, sema, semb):
    c = lax.axis_index("c")
    s = lax.axis_index("s")
    for i in range(16):
        for j in range(D // 16):
            zb[i, pl.ds(j * 16, 16)] = jnp.zeros((16,), jnp.float32)
    for i in range(K // 16):
        onev[pl.ds(i * 16, 16)] = jnp.ones((16,), jnp.float32)
    oner[...] = jnp.ones((REM,), jnp.float32)
    for i in range(ROWS_PER_TILE // 16):
        zv[pl.ds(i * 16, 16)] = jnp.zeros((16,), jnp.float32)
    for k in range(ROWS_PER_TILE // 16):
        pltpu.sync_copy(zb, acc.at[pl.ds(s * ROWS_PER_TILE + k * 16, 16)])
    pltpu.sync_copy(zv, hdeg.at[pl.ds(s * ROWS_PER_TILE, ROWS_PER_TILE)])
    plsc.subcore_barrier()
    base0 = (c * NS + s) * EPT

    def body(i, _):
        ba = base0 + 2 * i * K
        bb = ba + K
        pltpu.sync_copy(src.at[pl.ds(ba, K)], sia)
        pltpu.sync_copy(dst.at[pl.ds(ba, K)], dia)
        cpa = pltpu.async_copy(xn.at[sia], rowsa, sema)
        pltpu.sync_copy(src.at[pl.ds(bb, K)], sib)
        pltpu.sync_copy(dst.at[pl.ds(bb, K)], dib)
        cpb = pltpu.async_copy(xn.at[sib], rowsb, semb)
        pltpu.sync_copy(onev, hdeg.at[dia], add=True)
        cpa.wait()
        pltpu.sync_copy(rowsa, acc.at[dia], add=True)
        pltpu.sync_copy(onev, hdeg.at[dib], add=True)
        cpb.wait()
        pltpu.sync_copy(rowsb, acc.at[dib], add=True)
        return 0

    lax.fori_loop(0, NFULL // 2, body, 0)
    b = base0 + NFULL * K
    pltpu.sync_copy(src.at[pl.ds(b, REM)], sidr)
    pltpu.sync_copy(dst.at[pl.ds(b, REM)], didr)
    cpr = pltpu.async_copy(xn.at[sidr], rowr, sema)
    pltpu.sync_copy(oner, hdeg.at[didr], add=True)
    cpr.wait()
    pltpu.sync_copy(rowr, acc.at[didr], add=True)

    plsc.subcore_barrier()
    sl = pl.ds(s * ROWS_PER_TILE, ROWS_PER_TILE)
    pltpu.sync_copy(acc.at[sl], out.at[c, sl])
    pltpu.sync_copy(hdeg.at[sl], dout.at[c, sl])


_R = 1024
_GRID = NPAD // _R


def _pre1_body(g_ref, b_ref, wres_ref, wrel_ref, h_ref, proj_ref, xw_ref):
    x = h_ref[...]
    m = jnp.mean(x, axis=-1, keepdims=True)
    v = jnp.mean((x - m) * (x - m), axis=-1, keepdims=True)
    ln = (x - m) * lax.rsqrt(v + 1e-5) * g_ref[0, :] + b_ref[0, :]
    proj_ref[...] = jnp.dot(ln, wres_ref[...],
                            preferred_element_type=jnp.float32)
    xw_ref[...] = jnp.dot(x, wrel_ref[...],
                          preferred_element_type=jnp.float32)


_pre1_call = pl.pallas_call(
    _pre1_body,
    grid=(_GRID,),
    in_specs=[
        pl.BlockSpec((1, D), lambda i: (0, 0)),
        pl.BlockSpec((1, D), lambda i: (0, 0)),
        pl.BlockSpec((D, D), lambda i: (0, 0)),
        pl.BlockSpec((D, D), lambda i: (0, 0)),
        pl.BlockSpec((_R, D), lambda i: (i, 0)),
    ],
    out_specs=[
        pl.BlockSpec((_R, D), lambda i: (i, 0)),
        pl.BlockSpec((_R, D), lambda i: (i, 0)),
    ],
    out_shape=[
        jax.ShapeDtypeStruct((NPAD, D), jnp.float32),
        jax.ShapeDtypeStruct((NPAD, D), jnp.float32),
    ],
)


def _pre2_body(gate_ref, xw_ref, deg_ref, xn_ref):
    deg = deg_ref[0, :] + deg_ref[1, :]
    sc = jnp.where(deg > 0.0, lax.rsqrt(jnp.maximum(deg, 1.0)), 0.0)
    xn_ref[...] = xw_ref[...] * (sc * gate_ref[0, 0])[:, None]


_pre2_call = pl.pallas_call(
    _pre2_body,
    grid=(_GRID,),
    in_specs=[
        pl.BlockSpec(memory_space=pltpu.SMEM),
        pl.BlockSpec((_R, D), lambda i: (i, 0)),
        pl.BlockSpec((2, _R), lambda i: (0, i)),
    ],
    out_specs=pl.BlockSpec((_R, D), lambda i: (i, 0)),
    out_shape=jax.ShapeDtypeStruct((NPAD, D), jnp.float32),
)


def _upd_body(w1_ref, b1_ref, w2_ref, b2_ref, proj_ref, part_ref, deg_ref,
              out_ref):
    proj = proj_ref[...]
    deg = deg_ref[0, :] + deg_ref[1, :]
    sc = jnp.where(deg > 0.0, lax.rsqrt(jnp.maximum(deg, 1.0)), 0.0)
    agg = (part_ref[0] + part_ref[1]) * sc[:, None]
    u = jnp.concatenate([proj, agg], axis=-1)
    hpre = jnp.dot(u, w1_ref[...], preferred_element_type=jnp.float32)
    hpre = hpre + b1_ref[0, :]
    h = hpre * 0.5 * (1.0 + lax.erf(hpre * 0.7071067811865476))
    out = jnp.dot(h, w2_ref[...], preferred_element_type=jnp.float32)
    out_ref[...] = proj + out + b2_ref[0, :]


_upd_call = pl.pallas_call(
    _upd_body,
    grid=(_GRID,),
    in_specs=[
        pl.BlockSpec((2 * D, HID), lambda i: (0, 0)),
        pl.BlockSpec((1, HID), lambda i: (0, 0)),
        pl.BlockSpec((HID, D), lambda i: (0, 0)),
        pl.BlockSpec((1, D), lambda i: (0, 0)),
        pl.BlockSpec((_R, D), lambda i: (i, 0)),
        pl.BlockSpec((2, _R, D), lambda i: (0, i, 0)),
        pl.BlockSpec((2, _R), lambda i: (0, i)),
    ],
    out_specs=pl.BlockSpec((_R, D), lambda i: (i, 0)),
    out_shape=jax.ShapeDtypeStruct((NPAD, D), jnp.float32),
)


def kernel(h_V, h_E, edge_index_V_E, edge_index_E_V,
           ln_g_V, ln_b_V, W_res_V, W1_V, b1_V, W2_V, b2_V,
           ln_g_E, ln_b_E, W_res_E, W1_E, b1_E, W2_E, b2_E,
           W_rel_VE, W_rel_EV, gate_VE, gate_EV):
    src_VE = edge_index_V_E[0].astype(jnp.int32)
    dst_VE = edge_index_V_E[1].astype(jnp.int32)
    src_EV = edge_index_E_V[0].astype(jnp.int32)
    dst_EV = edge_index_E_V[1].astype(jnp.int32)

    degp = _deg_fn(src_VE, src_EV)

    pad = ((0, NPAD - N), (0, 0))
    hV = jnp.pad(h_V, pad)
    hE = jnp.pad(h_E, pad)
    g2 = lambda a: a.reshape(1, -1)
    gateVE = jnp.reshape(gate_VE, (1, 1))
    gateEV = jnp.reshape(gate_EV, (1, 1))

    proj_V, xw_VE = _pre1_call(g2(ln_g_V), g2(ln_b_V), W_res_V, W_rel_VE, hV)
    proj_E, xw_EV = _pre1_call(g2(ln_g_E), g2(ln_b_E), W_res_E, W_rel_EV, hE)
    xn_VE = _pre2_call(gateVE, xw_VE, degp[:, 0, :])
    xn_EV = _pre2_call(gateEV, xw_EV, degp[:, 1, :])

    part_E, degd_E = _spmm_fn(xn_VE, src_VE, dst_VE)
    part_V, degd_V = _spmm_fn(xn_EV, src_EV, dst_EV)

    out_E = _upd_call(W1_E, g2(b1_E), W2_E, g2(b2_E), proj_E, part_E, degd_E)
    out_V = _upd_call(W1_V, g2(b1_V), W2_V, g2(b2_V), proj_V, part_V, degd_V)
    return (out_V[:N], out_E[:N])


# speedup vs baseline: 1.0415x; 1.0133x over previous
import functools

import jax
import jax.numpy as jnp
from jax import lax
from jax.experimental import pallas as pl
from jax.experimental.pallas import tpu as pltpu
from jax.experimental.pallas import tpu_sc as plsc

N = 10000
NPAD = 10240
D = 128
HID = 256
NEDGE = 320000
NC = 2
NS = 16
EPT = NEDGE // (NC * NS)
K = 128
NFULL = EPT // K
REM = EPT - NFULL * K
ROWS_PER_TILE = NPAD // NS

_mesh = plsc.VectorSubcoreMesh(core_axis_name="c", subcore_axis_name="s")


@functools.partial(
    pl.kernel,
    out_type=jax.ShapeDtypeStruct((NC, 2, NPAD), jnp.float32),
    mesh=_mesh,
    scratch_types=[
        pltpu.VMEM((K,), jnp.int32),
        pltpu.VMEM((K,), jnp.int32),
        pltpu.VMEM((REM,), jnp.int32),
        pltpu.VMEM((K,), jnp.float32),
        pltpu.VMEM((REM,), jnp.float32),
        pltpu.VMEM((ROWS_PER_TILE,), jnp.float32),
        pltpu.VMEM_SHARED((NPAD,), jnp.float32),
        pltpu.VMEM_SHARED((NPAD,), jnp.float32),
        pltpu.SemaphoreType.DMA,
        pltpu.SemaphoreType.DMA,
    ],
)
def _deg_fn(i0, i1, out, idxa, idxb, idxr, onev, oner, zv, h0, h1, sa, sb):
    c = lax.axis_index("c")
    s = lax.axis_index("s")
    for i in range(K // 16):
        onev[pl.ds(i * 16, 16)] = jnp.ones((16,), jnp.float32)
    oner[...] = jnp.ones((REM,), jnp.float32)
    for i in range(ROWS_PER_TILE // 16):
        zv[pl.ds(i * 16, 16)] = jnp.zeros((16,), jnp.float32)
    for h in (h0, h1):
        pltpu.sync_copy(zv, h.at[pl.ds(s * ROWS_PER_TILE, ROWS_PER_TILE)])
    plsc.subcore_barrier()
    base0 = (c * NS + s) * EPT
    for src, h in ((i0, h0), (i1, h1)):
        def body(i, _, src=src, h=h):
            b = base0 + 2 * i * K
            cpa = pltpu.async_copy(src.at[pl.ds(b, K)], idxa, sa)
            cpb = pltpu.async_copy(src.at[pl.ds(b + K, K)], idxb, sb)
            cpa.wait()
            pltpu.sync_copy(onev, h.at[idxa], add=True)
            cpb.wait()
            pltpu.sync_copy(onev, h.at[idxb], add=True)
            return 0
        lax.fori_loop(0, NFULL // 2, body, 0)
        b = base0 + NFULL * K
        pltpu.sync_copy(src.at[pl.ds(b, REM)], idxr)
        pltpu.sync_copy(oner, h.at[idxr], add=True)
    plsc.subcore_barrier()
    sl = pl.ds(s * ROWS_PER_TILE, ROWS_PER_TILE)
    for r, h in enumerate((h0, h1)):
        pltpu.sync_copy(h.at[sl], out.at[c, r, sl])


@functools.partial(
    pl.kernel,
    out_type=[
        jax.ShapeDtypeStruct((NC, NPAD, D), jnp.float32),
        jax.ShapeDtypeStruct((NC, NPAD), jnp.float32),
    ],
    mesh=_mesh,
    scratch_types=[
        pltpu.VMEM((K,), jnp.int32),
        pltpu.VMEM((K,), jnp.int32),
        pltpu.VMEM((K,), jnp.int32),
        pltpu.VMEM((K,), jnp.int32),
        pltpu.VMEM((REM,), jnp.int32),
        pltpu.VMEM((REM,), jnp.int32),
        pltpu.VMEM((K, D), jnp.float32),
        pltpu.VMEM((K, D), jnp.float32),
        pltpu.VMEM((REM, D), jnp.float32),
        pltpu.VMEM((K,), jnp.float32),
        pltpu.VMEM((REM,), jnp.float32),
        pltpu.VMEM((16, D), jnp.float32),
        pltpu.VMEM((ROWS_PER_TILE,), jnp.float32),
        pltpu.VMEM_SHARED((NPAD, D), jnp.float32),
        pltpu.VMEM_SHARED((NPAD,), jnp.float32),
        pltpu.SemaphoreType.DMA,
        pltpu.SemaphoreType.DMA,
        pltpu.SemaphoreType.DMA,
        pltpu.SemaphoreType.DMA,
        pltpu.SemaphoreType.DMA,
        pltpu.SemaphoreType.DMA,
    ],
)
def _spmm_fn(xn, src, dst, out, dout,
             sia, dia, sib, dib, sidr, didr,
             rowsa, rowsb, rowr, onev, oner, zb, zv, acc, hdeg, sema, semb,
             si1, si2, si3, si4):
    c = lax.axis_index("c")
    s = lax.axis_index("s")
    for i in range(16):
        for j in range(D // 16):
            zb[i, pl.ds(j * 16, 16)] = jnp.zeros((16,), jnp.float32)
    for i in range(K // 16):
        onev[pl.ds(i * 16, 16)] = jnp.ones((16,), jnp.float32)
    oner[...] = jnp.ones((REM,), jnp.float32)
    for i in range(ROWS_PER_TILE // 16):
        zv[pl.ds(i * 16, 16)] = jnp.zeros((16,), jnp.float32)
    for k in range(ROWS_PER_TILE // 16):
        pltpu.sync_copy(zb, acc.at[pl.ds(s * ROWS_PER_TILE + k * 16, 16)])
    pltpu.sync_copy(zv, hdeg.at[pl.ds(s * ROWS_PER_TILE, ROWS_PER_TILE)])
    plsc.subcore_barrier()
    base0 = (c * NS + s) * EPT

    def body(i, _):
        ba = base0 + 2 * i * K
        bb = ba + K
        ca = pltpu.async_copy(src.at[pl.ds(ba, K)], sia, si1)
        cb = pltpu.async_copy(dst.at[pl.ds(ba, K)], dia, si2)
        cc = pltpu.async_copy(src.at[pl.ds(bb, K)], sib, si3)
        cd = pltpu.async_copy(dst.at[pl.ds(bb, K)], dib, si4)
        ca.wait()
        cpa = pltpu.async_copy(xn.at[sia], rowsa, sema)
        cc.wait()
        cpb = pltpu.async_copy(xn.at[sib], rowsb, semb)
        cb.wait()
        cd.wait()
        pltpu.sync_copy(onev, hdeg.at[dia], add=True)
        cpa.wait()
        pltpu.sync_copy(rowsa, acc.at[dia], add=True)
        pltpu.sync_copy(onev, hdeg.at[dib], add=True)
        cpb.wait()
        pltpu.sync_copy(rowsb, acc.at[dib], add=True)
        return 0

    lax.fori_loop(0, NFULL // 2, body, 0)
    b = base0 + NFULL * K
    pltpu.sync_copy(src.at[pl.ds(b, REM)], sidr)
    pltpu.sync_copy(dst.at[pl.ds(b, REM)], didr)
    cpr = pltpu.async_copy(xn.at[sidr], rowr, sema)
    pltpu.sync_copy(oner, hdeg.at[didr], add=True)
    cpr.wait()
    pltpu.sync_copy(rowr, acc.at[didr], add=True)

    plsc.subcore_barrier()
    sl = pl.ds(s * ROWS_PER_TILE, ROWS_PER_TILE)
    pltpu.sync_copy(acc.at[sl], out.at[c, sl])
    pltpu.sync_copy(hdeg.at[sl], dout.at[c, sl])


_R = 1024
_GRID = NPAD // _R


def _pre1_body(g_ref, b_ref, wres_ref, wrel_ref, h_ref, proj_ref, xw_ref):
    x = h_ref[...]
    m = jnp.mean(x, axis=-1, keepdims=True)
    v = jnp.mean((x - m) * (x - m), axis=-1, keepdims=True)
    ln = (x - m) * lax.rsqrt(v + 1e-5) * g_ref[0, :] + b_ref[0, :]
    proj_ref[...] = jnp.dot(ln, wres_ref[...],
                            preferred_element_type=jnp.float32)
    xw_ref[...] = jnp.dot(x, wrel_ref[...],
                          preferred_element_type=jnp.float32)


_pre1_call = pl.pallas_call(
    _pre1_body,
    grid=(_GRID,),
    in_specs=[
        pl.BlockSpec((1, D), lambda i: (0, 0)),
        pl.BlockSpec((1, D), lambda i: (0, 0)),
        pl.BlockSpec((D, D), lambda i: (0, 0)),
        pl.BlockSpec((D, D), lambda i: (0, 0)),
        pl.BlockSpec((_R, D), lambda i: (i, 0)),
    ],
    out_specs=[
        pl.BlockSpec((_R, D), lambda i: (i, 0)),
        pl.BlockSpec((_R, D), lambda i: (i, 0)),
    ],
    out_shape=[
        jax.ShapeDtypeStruct((NPAD, D), jnp.float32),
        jax.ShapeDtypeStruct((NPAD, D), jnp.float32),
    ],
)


def _pre2_body(gate_ref, xw_ref, deg_ref, xn_ref):
    deg = deg_ref[0, :] + deg_ref[1, :]
    sc = jnp.where(deg > 0.0, lax.rsqrt(jnp.maximum(deg, 1.0)), 0.0)
    xn_ref[...] = xw_ref[...] * (sc * gate_ref[0, 0])[:, None]


_pre2_call = pl.pallas_call(
    _pre2_body,
    grid=(_GRID,),
    in_specs=[
        pl.BlockSpec(memory_space=pltpu.SMEM),
        pl.BlockSpec((_R, D), lambda i: (i, 0)),
        pl.BlockSpec((2, _R), lambda i: (0, i)),
    ],
    out_specs=pl.BlockSpec((_R, D), lambda i: (i, 0)),
    out_shape=jax.ShapeDtypeStruct((NPAD, D), jnp.float32),
)


def _upd_body(w1_ref, b1_ref, w2_ref, b2_ref, proj_ref, part_ref, deg_ref,
              out_ref):
    proj = proj_ref[...]
    deg = deg_ref[0, :] + deg_ref[1, :]
    sc = jnp.where(deg > 0.0, lax.rsqrt(jnp.maximum(deg, 1.0)), 0.0)
    agg = (part_ref[0] + part_ref[1]) * sc[:, None]
    u = jnp.concatenate([proj, agg], axis=-1)
    hpre = jnp.dot(u, w1_ref[...], preferred_element_type=jnp.float32)
    hpre = hpre + b1_ref[0, :]
    h = hpre * 0.5 * (1.0 + lax.erf(hpre * 0.7071067811865476))
    out = jnp.dot(h, w2_ref[...], preferred_element_type=jnp.float32)
    out_ref[...] = proj + out + b2_ref[0, :]


_upd_call = pl.pallas_call(
    _upd_body,
    grid=(_GRID,),
    in_specs=[
        pl.BlockSpec((2 * D, HID), lambda i: (0, 0)),
        pl.BlockSpec((1, HID), lambda i: (0, 0)),
        pl.BlockSpec((HID, D), lambda i: (0, 0)),
        pl.BlockSpec((1, D), lambda i: (0, 0)),
        pl.BlockSpec((_R, D), lambda i: (i, 0)),
        pl.BlockSpec((2, _R, D), lambda i: (0, i, 0)),
        pl.BlockSpec((2, _R), lambda i: (0, i)),
    ],
    out_specs=pl.BlockSpec((_R, D), lambda i: (i, 0)),
    out_shape=jax.ShapeDtypeStruct((NPAD, D), jnp.float32),
)


def kernel(h_V, h_E, edge_index_V_E, edge_index_E_V,
           ln_g_V, ln_b_V, W_res_V, W1_V, b1_V, W2_V, b2_V,
           ln_g_E, ln_b_E, W_res_E, W1_E, b1_E, W2_E, b2_E,
           W_rel_VE, W_rel_EV, gate_VE, gate_EV):
    src_VE = edge_index_V_E[0].astype(jnp.int32)
    dst_VE = edge_index_V_E[1].astype(jnp.int32)
    src_EV = edge_index_E_V[0].astype(jnp.int32)
    dst_EV = edge_index_E_V[1].astype(jnp.int32)

    degp = _deg_fn(src_VE, src_EV)

    pad = ((0, NPAD - N), (0, 0))
    hV = jnp.pad(h_V, pad)
    hE = jnp.pad(h_E, pad)
    g2 = lambda a: a.reshape(1, -1)
    gateVE = jnp.reshape(gate_VE, (1, 1))
    gateEV = jnp.reshape(gate_EV, (1, 1))

    proj_V, xw_VE = _pre1_call(g2(ln_g_V), g2(ln_b_V), W_res_V, W_rel_VE, hV)
    proj_E, xw_EV = _pre1_call(g2(ln_g_E), g2(ln_b_E), W_res_E, W_rel_EV, hE)
    xn_VE = _pre2_call(gateVE, xw_VE, degp[:, 0, :])
    xn_EV = _pre2_call(gateEV, xw_EV, degp[:, 1, :])

    part_E, degd_E = _spmm_fn(xn_VE, src_VE, dst_VE)
    part_V, degd_V = _spmm_fn(xn_EV, src_EV, dst_EV)

    out_E = _upd_call(W1_E, g2(b1_E), W2_E, g2(b2_E), proj_E, part_E, degd_E)
    out_V = _upd_call(W1_V, g2(b1_V), W2_V, g2(b2_V), proj_V, part_V, degd_V)
    return (out_V[:N], out_E[:N])
